# baseline (speedup 1.0000x reference)
import functools

import jax
import jax.numpy as jnp
from jax import lax
from jax.experimental import pallas as pl
from jax.experimental.pallas import tpu as pltpu
from jax.experimental.pallas import tpu_sc as plsc

B = 4096
T = 192
F = 36
NUM = 10
NEMB = 26
V = 100000
D = 32
TIN = 168

NC, NS = 2, 16
NW = NC * NS
BPW = B // NW


def _emb_kernel(idx_hbm, tab_hbm, out_hbm, idx_v, rows_v, acc_v, sem):
    wid = lax.axis_index("s") * NC + lax.axis_index("c")
    base = wid * BPW
    pltpu.sync_copy(idx_hbm.at[:, pl.ds(base, BPW)], idx_v)
    copies = []
    for j in range(NEMB):
        copies.append(
            pltpu.async_copy(
                tab_hbm.at[idx_v.at[j]],
                rows_v.at[pl.ds(j * BPW, BPW)],
                sem,
            )
        )
    for c in copies:
        c.wait()

    def body(b, carry):
        a0 = rows_v[b, 0:16]
        a1 = rows_v[b, 16:32]
        for j in range(1, NEMB):
            a0 = a0 + rows_v[j * BPW + b, 0:16]
            a1 = a1 + rows_v[j * BPW + b, 16:32]
        acc_v[b, 0:16] = a0
        acc_v[b, 16:32] = a1
        return carry

    lax.fori_loop(0, BPW, body, 0)
    pltpu.sync_copy(acc_v, out_hbm.at[pl.ds(base, BPW)])


@functools.partial(jax.jit, static_argnames=())
def _emb(idx_t, tab_flat):
    mesh = plsc.VectorSubcoreMesh(core_axis_name="c", subcore_axis_name="s")
    f = pl.kernel(
        _emb_kernel,
        out_type=jax.ShapeDtypeStruct((B, D), jnp.float32),
        mesh=mesh,
        scratch_types=[
            pltpu.VMEM((NEMB, BPW), jnp.int32),
            pltpu.VMEM((NEMB * BPW, D), jnp.float32),
            pltpu.VMEM((BPW, D), jnp.float32),
            pltpu.SemaphoreType.DMA,
        ],
        compiler_params=pltpu.CompilerParams(use_tc_tiling_on_sc=False),
    )
    return f(idx_t, tab_flat)


_BB = 32


def _slice_kernel(x_ref, past_ref, fut_ref, mask_ref):
    xb = x_ref[...]
    past_ref[...] = xb[:, :TIN, :NUM]
    fut_ref[...] = xb[:, TIN:, :NUM]
    m = (xb[:, :TIN, :1] > 0).astype(jnp.float32)
    mask_ref[...] = jnp.transpose(m, (0, 2, 1))


def _slices(x):
    grid = (B // _BB,)
    return pl.pallas_call(
        _slice_kernel,
        grid=grid,
        in_specs=[pl.BlockSpec((_BB, T, F), lambda i: (i, 0, 0))],
        out_specs=(
            pl.BlockSpec((_BB, TIN, NUM), lambda i: (i, 0, 0)),
            pl.BlockSpec((_BB, T - TIN, NUM), lambda i: (i, 0, 0)),
            pl.BlockSpec((_BB, 1, TIN), lambda i: (i, 0, 0)),
        ),
        out_shape=(
            jax.ShapeDtypeStruct((B, TIN, NUM), jnp.float32),
            jax.ShapeDtypeStruct((B, T - TIN, NUM), jnp.float32),
            jax.ShapeDtypeStruct((B, 1, TIN), jnp.float32),
        ),
    )(x)


def kernel(x, tables):
    past, fut, maskf = _slices(x)
    idx = x[:, 0, NUM:NUM + NEMB].astype(jnp.int32)
    offs = (jnp.arange(NEMB, dtype=jnp.int32) * V)[:, None]
    idx_t = idx.T + offs
    emb = _emb(idx_t, tables.reshape(NEMB * V, D))
    mask_out = maskf.astype(jnp.bool_)
    return (past, fut, emb, mask_out)


# speedup vs baseline: 1.0000x; 1.0000x over previous
import functools

import jax
import jax.numpy as jnp
from jax import lax
from jax.experimental import pallas as pl
from jax.experimental.pallas import tpu as pltpu
from jax.experimental.pallas import tpu_sc as plsc

B = 4096
T = 192
F = 36
NUM = 10
NEMB = 26
V = 100000
D = 32
TIN = 168

NC, NS = 2, 16
NW = NC * NS
BPW = B // NW


def _emb_kernel(idx_hbm, tab_hbm, out_hbm, idx_v, rows_v, acc_v, sem):
    wid = lax.axis_index("s") * NC + lax.axis_index("c")
    base = wid * BPW
    pltpu.sync_copy(idx_hbm.at[:, pl.ds(base, BPW)], idx_v)
    copies = []
    for j in range(NEMB):
        copies.append(
            pltpu.async_copy(
                tab_hbm.at[j].at[idx_v.at[j]],
                rows_v.at[pl.ds(j * BPW, BPW)],
                sem,
            )
        )
    for c in copies:
        c.wait()

    def body(b, carry):
        a0 = rows_v[b, 0:16]
        a1 = rows_v[b, 16:32]
        for j in range(1, NEMB):
            a0 = a0 + rows_v[j * BPW + b, 0:16]
            a1 = a1 + rows_v[j * BPW + b, 16:32]
        acc_v[b, 0:16] = a0
        acc_v[b, 16:32] = a1
        return carry

    lax.fori_loop(0, BPW, body, 0)
    pltpu.sync_copy(acc_v, out_hbm.at[pl.ds(base, BPW)])


def _emb(idx_t, tables):
    mesh = plsc.VectorSubcoreMesh(core_axis_name="c", subcore_axis_name="s")
    f = pl.kernel(
        _emb_kernel,
        out_type=jax.ShapeDtypeStruct((B, D), jnp.float32),
        mesh=mesh,
        scratch_types=[
            pltpu.VMEM((NEMB, BPW), jnp.int32),
            pltpu.VMEM((NEMB * BPW, D), jnp.float32),
            pltpu.VMEM((BPW, D), jnp.float32),
            pltpu.SemaphoreType.DMA,
        ],
        compiler_params=pltpu.CompilerParams(use_tc_tiling_on_sc=False),
    )
    return f(idx_t, tables)


_BB = 32


def _slice_kernel(x_ref, past_ref, fut_ref, mask_ref):
    xb = x_ref[...]
    past_ref[...] = xb[:, :TIN, :NUM]
    fut_ref[...] = xb[:, TIN:, :NUM]
    m = (xb[:, :TIN, :1] > 0).astype(jnp.float32)
    mask_ref[...] = jnp.transpose(m, (0, 2, 1))


def _slices(x):
    grid = (B // _BB,)
    return pl.pallas_call(
        _slice_kernel,
        grid=grid,
        in_specs=[pl.BlockSpec((_BB, T, F), lambda i: (i, 0, 0))],
        out_specs=(
            pl.BlockSpec((_BB, TIN, NUM), lambda i: (i, 0, 0)),
            pl.BlockSpec((_BB, T - TIN, NUM), lambda i: (i, 0, 0)),
            pl.BlockSpec((_BB, 1, TIN), lambda i: (i, 0, 0)),
        ),
        out_shape=(
            jax.ShapeDtypeStruct((B, TIN, NUM), jnp.float32),
            jax.ShapeDtypeStruct((B, T - TIN, NUM), jnp.float32),
            jax.ShapeDtypeStruct((B, 1, TIN), jnp.float32),
        ),
    )(x)


def kernel(x, tables):
    past, fut, maskf = _slices(x)
    idx = x[:, 0, NUM:NUM + NEMB].astype(jnp.int32)
    idx_t = idx.T
    emb = _emb(idx_t, tables)
    mask_out = maskf.astype(jnp.bool_)
    return (past, fut, emb, mask_out)


# speedup vs baseline: 2.9718x; 2.9717x over previous
import jax
import jax.numpy as jnp
from jax import lax
from jax.experimental import pallas as pl
from jax.experimental.pallas import tpu as pltpu
from jax.experimental.pallas import tpu_sc as plsc

B = 4096
T = 192
F = 36
NUM = 10
NEMB = 26
V = 100000
D = 32
TIN = 168

NC, NS = 2, 16
NW = NC * NS
BPW = B // NW


def _emb_kernel(idx_hbm, tab_hbm, embt_hbm, idx_v, buf_v, acc_v, sem, gsem):
    wid = lax.axis_index("s") * NC + lax.axis_index("c")
    base = wid * BPW
    pltpu.sync_copy(idx_hbm.at[:, pl.ds(base, BPW)], idx_v)
    for j in range(NEMB):
        for d in range(D):
            pltpu.async_copy(
                tab_hbm.at[pl.ds((j * D + d) * V, V)].at[idx_v.at[j]],
                buf_v.at[j, d],
                gsem,
            )
    pltpu.make_async_copy(
        tab_hbm.at[pl.ds(0, NEMB * D * BPW)], buf_v, gsem
    ).wait()

    def body(j, carry):
        for d in range(D):
            for c in range(BPW // 16):
                s = pl.ds(c * 16, 16)
                acc_v[d, s] = acc_v[d, s] + buf_v[j, d, s]
        return carry

    for d in range(D):
        for c in range(BPW // 16):
            s = pl.ds(c * 16, 16)
            acc_v[d, s] = buf_v[0, d, s]
    lax.fori_loop(1, NEMB, body, 0)
    pltpu.sync_copy(acc_v, embt_hbm.at[:, pl.ds(base, BPW)])


def _emb(idx_t, tab_flat):
    mesh = plsc.VectorSubcoreMesh(core_axis_name="c", subcore_axis_name="s")
    f = pl.kernel(
        _emb_kernel,
        out_type=jax.ShapeDtypeStruct((D, B), jnp.float32),
        mesh=mesh,
        scratch_types=[
            pltpu.VMEM((NEMB, BPW), jnp.int32),
            pltpu.VMEM((NEMB, D, BPW), jnp.float32),
            pltpu.VMEM((D, BPW), jnp.float32),
            pltpu.SemaphoreType.DMA,
            pltpu.SemaphoreType.DMA,
        ],
        compiler_params=pltpu.CompilerParams(use_tc_tiling_on_sc=False),
    )
    return f(idx_t, tab_flat)


def _idx_kernel(x_ref, idx_ref):
    idx_ref[...] = x_ref[NUM:, 0, :].astype(jnp.int32)


def _idx(xT):
    return pl.pallas_call(
        _idx_kernel,
        grid=(1,),
        in_specs=[pl.BlockSpec((F, 8, B), lambda i: (0, 0, 0))],
        out_specs=pl.BlockSpec((NEMB, B), lambda i: (0, 0)),
        out_shape=jax.ShapeDtypeStruct((NEMB, B), jnp.int32),
    )(xT)


_BL = 512


def _slice_kernel(x_ref, past_ref, fut_ref, mask_ref):
    past_ref[...] = x_ref[:NUM, :TIN, :]
    fut_ref[...] = x_ref[:NUM, TIN:, :]
    mask_ref[...] = (x_ref[0, :TIN, :] > 0).astype(jnp.float32)


def _slices(xT):
    return pl.pallas_call(
        _slice_kernel,
        grid=(B // _BL,),
        in_specs=[pl.BlockSpec((F, T, _BL), lambda i: (0, 0, i))],
        out_specs=(
            pl.BlockSpec((NUM, TIN, _BL), lambda i: (0, 0, i)),
            pl.BlockSpec((NUM, T - TIN, _BL), lambda i: (0, 0, i)),
            pl.BlockSpec((TIN, _BL), lambda i: (0, i)),
        ),
        out_shape=(
            jax.ShapeDtypeStruct((NUM, TIN, B), jnp.float32),
            jax.ShapeDtypeStruct((NUM, T - TIN, B), jnp.float32),
            jax.ShapeDtypeStruct((TIN, B), jnp.float32),
        ),
    )(xT)


def kernel(x, tables):
    xT = x.transpose(2, 1, 0)
    pastT, futT, maskT = _slices(xT)
    past = pastT.transpose(2, 1, 0)
    fut = futT.transpose(2, 1, 0)
    mask_out = maskT.T.reshape(B, 1, TIN).astype(jnp.bool_)

    idx_t = _idx(xT)
    tab_flat = tables.transpose(0, 2, 1).reshape(NEMB * D * V)
    embT = _emb(idx_t, tab_flat)
    emb = embT.T
    return (past, fut, emb, mask_out)
